# Initial kernel scaffold; baseline (speedup 1.0000x reference)
import jax
import jax.numpy as jnp
from jax.experimental import pallas as pl


def kernel(drug1_id, drug2_id, ent_emb, src, dst, rel_id, rel_emb, W_edge, g_edge, b_edge, W_node, g_node, b_node, W_comp, g_comp, b_comp):
    raise NotImplementedError("write your pallas kernel here")


# speedup vs baseline: 2.4740x; 2.4740x over previous
import functools

import jax
import jax.numpy as jnp
from jax import lax
from jax.experimental import pallas as pl
from jax.experimental.pallas import tpu as pltpu
from jax.experimental.pallas import tpu_sc as plsc

N_ENT = 10000
N_EDGE = 320000
H = 128
EPS_BN = 1e-5
K = 80


def _row_chunks(total):
    out, off = [], 0
    while off < total:
        sz = min(K, total - off)
        out.append((off, sz))
        off += sz
    return out
RANGES = [(0, 3360), (3360, 3360), (6720, 3280)]
TRASH = 3360
ACC_ROWS = 3368


def _sc_passes(ent_emb, rel_emb, src, dst, rel_id):
    info = plsc.get_sparse_core_info()
    NC, NS = info.num_cores, info.num_subcores
    NW = NC * NS
    per_tile = N_EDGE // NW
    n_chunks = per_tile // K
    n_init_sub = 10
    mesh = plsc.VectorSubcoreMesh(core_axis_name="c", subcore_axis_name="s")
    acc_t = jax.ShapeDtypeStruct((NC, N_ENT, H), jnp.float32)

    @functools.partial(
        pl.kernel,
        mesh=mesh,
        out_type=[acc_t, acc_t, acc_t, acc_t],
        scratch_types=[
            pltpu.VMEM((K,), jnp.int32),
            pltpu.VMEM((K,), jnp.int32),
            pltpu.VMEM((K,), jnp.int32),
            pltpu.VMEM((K,), jnp.int32),
            pltpu.VMEM((K, H), jnp.float32),
            pltpu.VMEM((K, H), jnp.float32),
            pltpu.VMEM((K, H), jnp.float32),
            pltpu.VMEM((K, H), jnp.float32),
            pltpu.VMEM_SHARED((ACC_ROWS, H), jnp.float32),
            pltpu.SemaphoreType.DMA,
        ],
    )
    def sc_kernel(ent_hbm, rel_hbm, src_hbm, dst_hbm, rid_hbm,
                  o1, o2, o3, o4, si_v, di_v, dil_v, ri_v, u_v, v_v, r_v,
                  ow_v, acc_sh, sem):
        cid = lax.axis_index("c")
        sid = lax.axis_index("s")
        wid = sid * NC + cid
        base = wid * per_tile
        iota16 = lax.iota(jnp.int32, 16)
        zeros16 = jnp.zeros((16,), jnp.float32)
        dnums = lax.GatherDimensionNumbers(
            offset_dims=(), collapsed_slice_dims=(0,), start_index_map=(0,))

        def lanesum(s):
            for shift in (8, 4, 2, 1):
                perm = jnp.bitwise_xor(iota16, shift)
                s = s + lax.gather(
                    s, perm.reshape(16, 1), dnums, (1,),
                    mode=lax.GatherScatterMode.PROMISE_IN_BOUNDS)
            return s

        def run_sweep(mode, out_hbm, base_row, n_rows):
            plsc.subcore_barrier()

            def zrow(i, zc):
                for c in range(H // 16):
                    ow_v[i, pl.ds(c * 16, 16)] = zeros16
                return zc

            lax.fori_loop(0, K, zrow, 0)

            def zbody(t, zc):
                roff = pl.multiple_of(t * K, 8)

                @pl.when(sid == lax.rem(t, NS))
                def _():
                    pltpu.sync_copy(ow_v, acc_sh.at[pl.ds(roff, K)])

                return zc

            lax.fori_loop(0, RANGES[0][1] // K, zbody, 0)

            @pl.when(sid == NS - 1)
            def _():
                pltpu.sync_copy(ow_v.at[pl.ds(0, ACC_ROWS - TRASH)],
                                acc_sh.at[pl.ds(TRASH, ACC_ROWS - TRASH)])

            plsc.subcore_barrier()

            need_u = mode != 0
            need_r = mode != 1

            def chunk_body(j, carry):
                off = base + j * K
                pltpu.sync_copy(dst_hbm.at[pl.ds(off, K)], di_v)
                pltpu.async_copy(ent_hbm.at[di_v], v_v, sem).wait()
                for g in range(K // 16):
                    sl16 = pl.ds(g * 16, 16)
                    d16 = di_v[sl16]
                    loc = d16 - base_row
                    ok = (loc >= 0) & (loc < n_rows)
                    dil_v[sl16] = jnp.where(ok, loc, TRASH)
                if need_u:
                    pltpu.sync_copy(src_hbm.at[pl.ds(off, K)], si_v)
                    pltpu.async_copy(ent_hbm.at[si_v], u_v, sem).wait()
                if need_r:
                    pltpu.sync_copy(rid_hbm.at[pl.ds(off, K)], ri_v)
                    pltpu.async_copy(rel_hbm.at[ri_v], r_v, sem).wait()

                def edge_body(i, ecarry):
                    if mode == 3:
                        s1 = s2 = s3 = zeros16
                        for c in range(H // 16):
                            sl = pl.ds(c * 16, 16)
                            vv = v_v[i, sl]
                            uu = u_v[i, sl]
                            rr = r_v[i, sl]
                            s1 = s1 + rr * vv
                            s2 = s2 + uu * vv
                            s3 = s3 + uu * rr * vv
                        ow_v[i, pl.ds(0, 16)] = jnp.exp(lanesum(s1))
                        ow_v[i, pl.ds(16, 16)] = jnp.exp(lanesum(s2))
                        ow_v[i, pl.ds(32, 16)] = jnp.exp(lanesum(s3))
                        for c in range(3, H // 16):
                            ow_v[i, pl.ds(c * 16, 16)] = zeros16
                    else:
                        vals = []
                        s = zeros16
                        for c in range(H // 16):
                            sl = pl.ds(c * 16, 16)
                            vv = v_v[i, sl]
                            if mode == 0:
                                val = r_v[i, sl]
                            elif mode == 1:
                                val = u_v[i, sl]
                            else:
                                val = u_v[i, sl] * r_v[i, sl]
                            vals.append(val)
                            s = s + val * vv
                        ex = jnp.exp(lanesum(s))
                        for c in range(H // 16):
                            ow_v[i, pl.ds(c * 16, 16)] = vals[c] * ex
                    return ecarry

                lax.fori_loop(0, K, edge_body, 0)
                pltpu.sync_copy(ow_v, acc_sh.at[dil_v], add=True)
                return carry

            lax.fori_loop(0, n_chunks, chunk_body, 0)
            plsc.subcore_barrier()

            def dbody(t, dc):
                roff = pl.multiple_of(t * K, 8)
                goff = pl.multiple_of(base_row + t * K, 8)

                @pl.when(sid == lax.rem(t, NS))
                def _():
                    pltpu.sync_copy(acc_sh.at[pl.ds(roff, K)],
                                    out_hbm.at[cid, pl.ds(goff, K)])

                return dc

            lax.fori_loop(0, n_rows // K, dbody, 0)

        for mode, out in ((0, o1), (1, o2), (2, o3), (3, o4)):
            for base_row, n_rows in RANGES:
                run_sweep(mode, out, base_row, n_rows)
        plsc.subcore_barrier()

    return sc_kernel(ent_emb, rel_emb, src, dst, rel_id)


def _tc_layer(num, dens, lane, ent_emb, W, g, b):
    def body(nn, dd, ent, w, gg, bb, out):
        ent_v = ent[...]
        x_min = jnp.min(ent_v, axis=0, keepdims=True)
        x_max = jnp.max(ent_v, axis=0, keepdims=True)
        rng = x_max - x_min
        neigh = nn[...] / (dd[...][:, lane:lane + 1] + 1e-16)
        y = jnp.dot(neigh, w[...], preferred_element_type=jnp.float32,
                    precision=lax.Precision.HIGHEST)
        mu = jnp.mean(y, axis=0, keepdims=True)
        var = jnp.mean((y - mu) ** 2, axis=0, keepdims=True)
        h = jnp.tanh((y - mu) / jnp.sqrt(var + EPS_BN) * gg[...] + bb[...])
        out[...] = (h - x_min) / rng

    return pl.pallas_call(
        body,
        out_shape=jax.ShapeDtypeStruct((N_ENT, H), jnp.float32),
    )(num, dens, ent_emb, W, g, b)


def _tc_gather(ent_emb, h1, h2, h3, i1, i2):
    nb = i1.shape[0]
    CH = 1000

    def body(ent, a, bb, c3, id1, id2, out1, out2):
        def gather_chunk(c, carry):
            a1, a2 = carry
            sl = pl.ds(c * CH, CH)
            blk = ent[sl, :] + a[sl, :] + bb[sl, :] + c3[sl, :]
            node_ids = (lax.broadcasted_iota(jnp.int32, (nb, CH), 1)
                        + c * CH)
            oh1 = (id1[...] == node_ids).astype(jnp.float32)
            oh2 = (id2[...] == node_ids).astype(jnp.float32)
            a1 = a1 + jnp.dot(oh1, blk, preferred_element_type=jnp.float32,
                              precision=lax.Precision.HIGHEST)
            a2 = a2 + jnp.dot(oh2, blk, preferred_element_type=jnp.float32,
                              precision=lax.Precision.HIGHEST)
            return (a1, a2)

        z = jnp.zeros((nb, H), jnp.float32)
        a1, a2 = lax.fori_loop(0, N_ENT // CH, gather_chunk, (z, z))
        out1[...] = a1
        out2[...] = a2

    return pl.pallas_call(
        body,
        out_shape=[jax.ShapeDtypeStruct((nb, H), jnp.float32)] * 2,
    )(ent_emb, h1, h2, h3, i1, i2)


def kernel(drug1_id, drug2_id, ent_emb, src, dst, rel_id, rel_emb,
           W_edge, g_edge, b_edge, W_node, g_node, b_node,
           W_comp, g_comp, b_comp):
    src = src.astype(jnp.int32)
    dst = dst.astype(jnp.int32)
    rel_id = rel_id.astype(jnp.int32)

    o1, o2, o3, o4 = _sc_passes(ent_emb, rel_emb, src, dst, rel_id)

    num1 = o1[0] + o1[1]
    num2 = o2[0] + o2[1]
    num3 = o3[0] + o3[1]
    dens = o4[0] + o4[1]

    i1 = drug1_id.astype(jnp.int32).reshape(-1, 1)
    i2 = drug2_id.astype(jnp.int32).reshape(-1, 1)
    g_e = g_edge.reshape(1, H); b_e = b_edge.reshape(1, H)
    g_n = g_node.reshape(1, H); b_n = b_node.reshape(1, H)
    g_c = g_comp.reshape(1, H); b_c = b_comp.reshape(1, H)

    h1 = _tc_layer(num1, dens, 0, ent_emb, W_edge, g_e, b_e)
    h2 = _tc_layer(num2, dens, 16, ent_emb, W_node, g_n, b_n)
    h3 = _tc_layer(num3, dens, 32, ent_emb, W_comp, g_c, b_c)
    return _tc_gather(ent_emb, h1, h2, h3, i1, i2)
